# baseline (speedup 1.0000x reference)
import functools

import jax
import jax.numpy as jnp
from jax import lax
from jax.experimental import pallas as pl
from jax.experimental.pallas import tpu as pltpu
from jax.experimental.pallas import tpu_sc as plsc

_DIM = 64
_NC = 2
_NS = 16
_NW = _NC * _NS
_CHUNK = 128


@functools.lru_cache(maxsize=None)
def _build(total_rows: int, vocab: int):
    assert total_rows % (_NW * _CHUNK) == 0
    b_per_w = total_rows // _NW
    nchunks = b_per_w // _CHUNK
    mesh = plsc.VectorSubcoreMesh(core_axis_name="c", subcore_axis_name="s")

    @functools.partial(
        pl.kernel,
        mesh=mesh,
        out_type=jax.ShapeDtypeStruct((total_rows, _DIM), jnp.float32),
        scratch_types=[
            pltpu.VMEM((_CHUNK,), jnp.int32),
            pltpu.VMEM((_CHUNK, _DIM), jnp.float32),
            pltpu.SemaphoreType.DMA,
        ],
        compiler_params=pltpu.CompilerParams(use_tc_tiling_on_sc=False),
    )
    def emb(x_hbm, table_hbm, out_hbm, idx_v, rows_v, sem):
        wid = lax.axis_index("s") * _NC + lax.axis_index("c")
        base = wid * b_per_w

        def body(i, carry):
            off = base + i * _CHUNK
            pltpu.sync_copy(x_hbm.at[pl.ds(off, _CHUNK)], idx_v)
            pltpu.async_copy(table_hbm.at[idx_v], rows_v, sem).wait()
            pltpu.sync_copy(rows_v, out_hbm.at[pl.ds(off, _CHUNK)])
            return carry

        lax.fori_loop(0, nchunks, body, 0)

    return emb


def kernel(x, table):
    batch, hist = x.shape
    total = batch * hist
    flat = x.reshape(total).astype(jnp.int32)
    out = _build(total, table.shape[0])(flat, table)
    return out.reshape(batch, hist, _DIM)


# speedup vs baseline: 1.0720x; 1.0720x over previous
import functools

import jax
import jax.numpy as jnp
from jax import lax
from jax.experimental import pallas as pl
from jax.experimental.pallas import tpu as pltpu
from jax.experimental.pallas import tpu_sc as plsc

_DIM = 64
_NC = 2
_NS = 16
_NW = _NC * _NS
_CHUNK = 128
_NBUF = 5


@functools.lru_cache(maxsize=None)
def _build(total_rows: int, vocab: int):
    assert total_rows % (_NW * _CHUNK) == 0
    b_per_w = total_rows // _NW
    nchunks = b_per_w // _CHUNK
    assert nchunks % _NBUF == 0 and nchunks // _NBUF >= 2
    ngroups = nchunks // _NBUF - 1
    mesh = plsc.VectorSubcoreMesh(core_axis_name="c", subcore_axis_name="s")

    @functools.partial(
        pl.kernel,
        mesh=mesh,
        out_type=jax.ShapeDtypeStruct((total_rows, _DIM), jnp.float32),
        scratch_types=[
            [pltpu.VMEM((_CHUNK,), jnp.int32) for _ in range(_NBUF)],
            [pltpu.VMEM((_CHUNK, _DIM), jnp.float32) for _ in range(_NBUF)],
            [pltpu.SemaphoreType.DMA for _ in range(_NBUF)],
            [pltpu.SemaphoreType.DMA for _ in range(_NBUF)],
        ],
        compiler_params=pltpu.CompilerParams(use_tc_tiling_on_sc=False),
    )
    def emb(x_hbm, table_hbm, out_hbm, idx, rows, sem_g, sem_w):
        wid = lax.axis_index("s") * _NC + lax.axis_index("c")
        base = wid * b_per_w

        for b in range(_NBUF):
            pltpu.sync_copy(x_hbm.at[pl.ds(base + b * _CHUNK, _CHUNK)], idx[b])
            pltpu.async_copy(table_hbm.at[idx[b]], rows[b], sem_g[b])

        def body(g, carry):
            for b in range(_NBUF):
                i_w = g * _NBUF + b
                i_n = i_w + _NBUF
                pltpu.make_async_copy(table_hbm.at[idx[b]], rows[b], sem_g[b]).wait()
                w = pltpu.async_copy(
                    rows[b], out_hbm.at[pl.ds(base + i_w * _CHUNK, _CHUNK)], sem_w[b])
                pltpu.sync_copy(x_hbm.at[pl.ds(base + i_n * _CHUNK, _CHUNK)], idx[b])
                w.wait()
                pltpu.async_copy(table_hbm.at[idx[b]], rows[b], sem_g[b])
            return carry

        lax.fori_loop(0, ngroups, body, 0)

        last = ngroups * _NBUF
        for b in range(_NBUF):
            pltpu.make_async_copy(table_hbm.at[idx[b]], rows[b], sem_g[b]).wait()
            pltpu.async_copy(
                rows[b], out_hbm.at[pl.ds(base + (last + b) * _CHUNK, _CHUNK)], sem_w[b])
        for b in range(_NBUF):
            pltpu.make_async_copy(
                rows[b], out_hbm.at[pl.ds(base + (last + b) * _CHUNK, _CHUNK)], sem_w[b]
            ).wait()

    return emb


def kernel(x, table):
    batch, hist = x.shape
    total = batch * hist
    flat = x.reshape(total).astype(jnp.int32)
    out = _build(total, table.shape[0])(flat, table)
    return out.reshape(batch, hist, _DIM)
